# Initial kernel scaffold; baseline (speedup 1.0000x reference)
import jax
import jax.numpy as jnp
from jax.experimental import pallas as pl


def kernel(state, edge_index, W1l, W1r, b1, W2l, W2r, b2, W4, b4, W5, b5):
    raise NotImplementedError("write your pallas kernel here")


# speedup vs baseline: 1.5299x; 1.5299x over previous
import functools

import jax
import jax.numpy as jnp
from jax import lax
from jax.experimental import pallas as pl
from jax.experimental.pallas import tpu as pltpu
from jax.experimental.pallas import tpu_sc as plsc

N = 10000
E = 320000
D = 128

NW = 32
RPT = 313
NP = NW * RPT
CH = 4000
GH = CH // 16
B = 128

_NEG = float("-inf")


def _make_segmax(Df):
    KV = Df // 16
    mesh = plsc.VectorSubcoreMesh(core_axis_name="c", subcore_axis_name="s")

    @functools.partial(
        pl.kernel,
        out_type=jax.ShapeDtypeStruct((NP * Df,), jnp.float32),
        mesh=mesh,
        compiler_params=pltpu.CompilerParams(
            needs_layout_passes=False, use_tc_tiling_on_sc=False),
        scratch_types=[
            pltpu.VMEM((CH,), jnp.int32),
            pltpu.VMEM((CH,), jnp.int32),
            pltpu.VMEM((B + 16,), jnp.int32),
            pltpu.VMEM((B + 16,), jnp.int32),
            pltpu.VMEM((B, Df), jnp.float32),
            pltpu.VMEM(((RPT + 1) * Df,), jnp.float32),
            pltpu.SemaphoreType.DMA,
        ],
    )
    def seg(x_hbm, src_hbm, dst_hbm, out_hbm, dstv, srcv, selsrc, seldl,
            rows, acc, sem):
        wid = lax.axis_index("s") * 2 + lax.axis_index("c")
        lo = (wid * RPT).astype(jnp.int32)

        def init_body(i, _):
            acc[pl.ds(i * 16, 16)] = jnp.full((16,), _NEG, jnp.float32)
            return 0
        lax.fori_loop(0, (RPT + 1) * Df // 16, init_body, 0)

        def flush(cnt):
            pltpu.async_copy(x_hbm.at[selsrc.at[pl.ds(0, B)]], rows, sem).wait()

            def merge(e, _):
                base = seldl[pl.ds(e, 16)][0] * Df
                for k in range(KV):
                    a = acc[pl.ds(base + k * 16, 16)]
                    r = rows[e, pl.ds(k * 16, 16)]
                    acc[pl.ds(base + k * 16, 16)] = jnp.maximum(a, r)
                return 0
            lax.fori_loop(0, B, merge, 0)
            selsrc[pl.ds(0, 16)] = selsrc[pl.ds(B, 16)]
            seldl[pl.ds(0, 16)] = seldl[pl.ds(B, 16)]
            return cnt - B

        def chunk_body(c, cnt):
            pltpu.sync_copy(dst_hbm.at[pl.ds(c * CH, CH)], dstv)
            pltpu.sync_copy(src_hbm.at[pl.ds(c * CH, CH)], srcv)

            def group_body(g, cnt):
                d = dstv[pl.ds(g * 16, 16)]
                s = srcv[pl.ds(g * 16, 16)]
                dl = d - lo
                m = (dl >= 0) & (dl < RPT)
                mi = m.astype(jnp.int32)
                cs = plsc.cumsum(mi)
                pos = cnt + cs - 1
                plsc.store_scatter(selsrc, [pos], s, mask=m)
                plsc.store_scatter(seldl, [pos], dl, mask=m)
                cnt = cnt + cs[15]
                return lax.cond(cnt >= B, flush, lambda c: c, cnt)
            return lax.fori_loop(0, GH, group_body, cnt)

        cnt = lax.fori_loop(0, E // CH, chunk_body, jnp.int32(0))

        def pad_body(j, _):
            p = jnp.minimum(cnt + j * 16, jnp.int32(B))
            selsrc[pl.ds(p, 16)] = jnp.zeros((16,), jnp.int32)
            seldl[pl.ds(p, 16)] = jnp.full((16,), RPT, jnp.int32)
            return 0
        lax.fori_loop(0, B // 16 + 1, pad_body, 0)
        flush(cnt)

        def fix_body(i, _):
            v = acc[pl.ds(i * 16, 16)]
            acc[pl.ds(i * 16, 16)] = jnp.where(v == _NEG, jnp.float32(0.0), v)
            return 0
        lax.fori_loop(0, RPT * Df // 16, fix_body, 0)
        pltpu.sync_copy(acc.at[pl.ds(0, RPT * Df)],
                        out_hbm.at[pl.ds(lo * Df, RPT * Df)])

    return seg


_seg128 = _make_segmax(128)
_seg16 = _make_segmax(16)


def _tc1_body(a_ref, x_ref, wl_ref, wr_ref, b_ref, o_ref):
    o_ref[...] = jax.nn.relu(
        jnp.dot(a_ref[...], wl_ref[...], preferred_element_type=jnp.float32)
        + jnp.dot(x_ref[...], wr_ref[...], preferred_element_type=jnp.float32)
        + b_ref[...])


def _tc2_body(a_ref, h_ref, wl_ref, wr_ref, b2_ref, w4_ref, b4_ref, w5_ref,
              b5_ref, o_ref):
    h2 = jax.nn.relu(
        jnp.dot(a_ref[...], wl_ref[...], preferred_element_type=jnp.float32)
        + jnp.dot(h_ref[...], wr_ref[...], preferred_element_type=jnp.float32)
        + b2_ref[...])
    h3 = jax.nn.relu(
        jnp.dot(h2, w4_ref[...], preferred_element_type=jnp.float32)
        + b4_ref[...])
    o_ref[...] = (jnp.dot(h3, w5_ref[...], preferred_element_type=jnp.float32)
                  + b5_ref[...])


def kernel(state, edge_index, W1l, W1r, b1, W2l, W2r, b2, W4, b4, W5, b5):
    src = edge_index[0]
    dst = edge_index[1]
    xp = jnp.pad(state, ((0, NP - N), (0, 0)))

    agg1 = _seg128(xp, src, dst).reshape(NP, D)

    W1lp = jnp.pad(W1l, ((0, 0), (0, 2)))
    W1rp = jnp.pad(W1r, ((0, 0), (0, 2)))
    b1p = jnp.pad(b1, (0, 2)).reshape(1, 16)
    h1 = pl.pallas_call(
        _tc1_body,
        out_shape=jax.ShapeDtypeStruct((NP, 16), jnp.float32),
    )(agg1, xp, W1lp, W1rp, b1p)

    agg2 = _seg16(h1, src, dst).reshape(NP, 16)

    W2lp = jnp.pad(W2l, ((0, 2), (0, 0)))
    W2rp = jnp.pad(W2r, ((0, 2), (0, 0)))
    out = pl.pallas_call(
        _tc2_body,
        out_shape=jax.ShapeDtypeStruct((NP, 1), jnp.float32),
    )(agg2, h1, W2lp, W2rp, b2.reshape(1, 8), W4, b4.reshape(1, 5), W5,
      b5.reshape(1, 1))

    return out[:N, 0]


# speedup vs baseline: 1.8953x; 1.2388x over previous
import functools

import jax
import jax.numpy as jnp
from jax import lax
from jax.experimental import pallas as pl
from jax.experimental.pallas import tpu as pltpu
from jax.experimental.pallas import tpu_sc as plsc

N = 10000
E = 320000
D = 128

NW = 32
RPT = 313
NP = NW * RPT
CH = 4000
GH = CH // 16
NCH = E // CH
B = 128
CAP = (E // B + 1) * B

_NEG = float("-inf")

_params = pltpu.CompilerParams(needs_layout_passes=False,
                               use_tc_tiling_on_sc=False)
_mesh = plsc.VectorSubcoreMesh(core_axis_name="c", subcore_axis_name="s")


def _init_acc(acc, nvec):
    def body(i, _):
        acc[pl.ds(i * 16, 16)] = jnp.full((16,), _NEG, jnp.float32)
        return 0
    lax.fori_loop(0, nvec, body, 0)


def _merge_batch(x_hbm, idx_ref, dl_ref, rows, acc, sem, Df):
    pltpu.async_copy(x_hbm.at[idx_ref], rows, sem).wait()

    def merge(e, _):
        base = dl_ref[pl.ds(e, 16)][0] * Df
        for k in range(Df // 16):
            a = acc[pl.ds(base + k * 16, 16)]
            r = rows[e, pl.ds(k * 16, 16)]
            acc[pl.ds(base + k * 16, 16)] = jnp.maximum(a, r)
        return 0
    lax.fori_loop(0, B, merge, 0)


def _finish(acc, out_hbm, lo, Df):
    def fix(i, _):
        v = acc[pl.ds(i * 16, 16)]
        acc[pl.ds(i * 16, 16)] = jnp.where(v == _NEG, jnp.float32(0.0), v)
        return 0
    lax.fori_loop(0, RPT * Df // 16, fix, 0)
    pltpu.sync_copy(acc.at[pl.ds(0, RPT * Df)],
                    out_hbm.at[pl.ds(lo * Df, RPT * Df)])


@functools.partial(
    pl.kernel,
    out_type=(
        jax.ShapeDtypeStruct((NP * D,), jnp.float32),
        jax.ShapeDtypeStruct((NW * CAP,), jnp.int32),
        jax.ShapeDtypeStruct((NW * CAP,), jnp.int32),
        jax.ShapeDtypeStruct((NW * 16,), jnp.int32),
    ),
    mesh=_mesh,
    compiler_params=_params,
    scratch_types=[
        pltpu.VMEM((CH,), jnp.int32),
        pltpu.VMEM((CH,), jnp.int32),
        pltpu.VMEM((CH,), jnp.int32),
        pltpu.VMEM((CH,), jnp.int32),
        pltpu.VMEM((B + 16,), jnp.int32),
        pltpu.VMEM((B + 16,), jnp.int32),
        pltpu.VMEM((B, D), jnp.float32),
        pltpu.VMEM(((RPT + 1) * D,), jnp.float32),
        pltpu.SemaphoreType.DMA,
        pltpu.SemaphoreType.DMA,
        pltpu.SemaphoreType.DMA,
    ],
)
def _seg_scan(x_hbm, src_hbm, dst_hbm, out_hbm, lsrc_hbm, ldl_hbm, nb_hbm,
              dstv0, srcv0, dstv1, srcv1, selsrc, seldl, rows, acc,
              sem0, sem1, semg):
    wid = lax.axis_index("s") * 2 + lax.axis_index("c")
    lo = (wid * RPT).astype(jnp.int32)
    lbase = wid * CAP

    _init_acc(acc, (RPT + 1) * D // 16)

    def flush(carry):
        cnt, nb = carry
        _merge_batch(x_hbm, selsrc.at[pl.ds(0, B)], seldl, rows, acc, semg, D)
        pltpu.sync_copy(selsrc.at[pl.ds(0, B)],
                        lsrc_hbm.at[pl.ds(lbase + nb * B, B)])
        pltpu.sync_copy(seldl.at[pl.ds(0, B)],
                        ldl_hbm.at[pl.ds(lbase + nb * B, B)])
        selsrc[pl.ds(0, 16)] = selsrc[pl.ds(B, 16)]
        seldl[pl.ds(0, 16)] = seldl[pl.ds(B, 16)]
        return cnt - B, nb + 1

    def group_body(g, carry, dv, sv):
        d = dv[pl.ds(g * 16, 16)]
        s = sv[pl.ds(g * 16, 16)]
        dl = d - lo
        m = (dl >= 0) & (dl < RPT)
        pc = plsc.all_reduce_population_count(m)
        pcs = pc if pc.ndim == 0 else pc[0]

        def do(carry):
            cnt, nb = carry
            cs = plsc.cumsum(m.astype(jnp.int32))
            pos = cnt + cs - 1
            plsc.store_scatter(selsrc, [pos], s, mask=m)
            plsc.store_scatter(seldl, [pos], dl, mask=m)
            cnt = cnt + cs[15]
            return lax.cond(cnt >= B, flush, lambda c: c, (cnt, nb))
        return lax.cond(pcs > 0, do, lambda c: c, carry)

    def process(ci, nci, dv, sv, sem, carry):
        pltpu.make_async_copy(dst_hbm.at[pl.ds(0, CH)], dv, sem).wait()
        pltpu.make_async_copy(src_hbm.at[pl.ds(0, CH)], sv, sem).wait()
        carry = lax.fori_loop(
            0, GH, lambda g, c: group_body(g, c, dv, sv), carry)

        @pl.when(nci < NCH)
        def _():
            pltpu.async_copy(dst_hbm.at[pl.ds(nci * CH, CH)], dv, sem)
            pltpu.async_copy(src_hbm.at[pl.ds(nci * CH, CH)], sv, sem)
        return carry

    pltpu.async_copy(dst_hbm.at[pl.ds(0, CH)], dstv0, sem0)
    pltpu.async_copy(src_hbm.at[pl.ds(0, CH)], srcv0, sem0)
    pltpu.async_copy(dst_hbm.at[pl.ds(CH, CH)], dstv1, sem1)
    pltpu.async_copy(src_hbm.at[pl.ds(CH, CH)], srcv1, sem1)

    def pair_body(i, carry):
        carry = process(2 * i, 2 * i + 2, dstv0, srcv0, sem0, carry)
        carry = process(2 * i + 1, 2 * i + 3, dstv1, srcv1, sem1, carry)
        return carry

    cnt, nb = lax.fori_loop(0, NCH // 2, pair_body,
                            (jnp.int32(0), jnp.int32(0)))

    def pad_body(j, _):
        p = jnp.minimum(cnt + j * 16, jnp.int32(B))
        selsrc[pl.ds(p, 16)] = jnp.zeros((16,), jnp.int32)
        seldl[pl.ds(p, 16)] = jnp.full((16,), RPT, jnp.int32)
        return 0
    lax.fori_loop(0, B // 16 + 1, pad_body, 0)
    cnt, nb = flush((cnt, nb))

    selsrc[pl.ds(0, 16)] = jnp.broadcast_to(nb, (16,))
    pltpu.sync_copy(selsrc.at[pl.ds(0, 16)], nb_hbm.at[pl.ds(wid * 16, 16)])

    _finish(acc, out_hbm, lo, D)


@functools.partial(
    pl.kernel,
    out_type=jax.ShapeDtypeStruct((NP * 16,), jnp.float32),
    mesh=_mesh,
    compiler_params=_params,
    scratch_types=[
        pltpu.VMEM((B,), jnp.int32),
        pltpu.VMEM((B + 16,), jnp.int32),
        pltpu.VMEM((B, 16), jnp.float32),
        pltpu.VMEM(((RPT + 1) * 16,), jnp.float32),
        pltpu.VMEM((NW * 16,), jnp.int32),
        pltpu.SemaphoreType.DMA,
        pltpu.SemaphoreType.DMA,
    ],
)
def _seg_replay(x_hbm, lsrc_hbm, ldl_hbm, nb_hbm, out_hbm,
                selsrc, seldl, rows, acc, nbuf, sem, semg):
    wid = lax.axis_index("s") * 2 + lax.axis_index("c")
    lo = (wid * RPT).astype(jnp.int32)
    lbase = wid * CAP

    _init_acc(acc, (RPT + 1) * 16 // 16)
    pltpu.sync_copy(nb_hbm, nbuf)
    nb = nbuf[pl.ds(wid * 16, 16)][0]

    def batch(b, _):
        pltpu.sync_copy(lsrc_hbm.at[pl.ds(lbase + b * B, B)], selsrc)
        pltpu.sync_copy(ldl_hbm.at[pl.ds(lbase + b * B, B)],
                        seldl.at[pl.ds(0, B)])
        _merge_batch(x_hbm, selsrc, seldl, rows, acc, semg, 16)
        return 0
    lax.fori_loop(0, nb, batch, 0)

    _finish(acc, out_hbm, lo, 16)


def _tc1_body(a_ref, x_ref, wl_ref, wr_ref, b_ref, o_ref):
    o_ref[...] = jax.nn.relu(
        jnp.dot(a_ref[...], wl_ref[...], preferred_element_type=jnp.float32)
        + jnp.dot(x_ref[...], wr_ref[...], preferred_element_type=jnp.float32)
        + b_ref[...])


def _tc2_body(a_ref, h_ref, wl_ref, wr_ref, b2_ref, w4_ref, b4_ref, w5_ref,
              b5_ref, o_ref):
    h2 = jax.nn.relu(
        jnp.dot(a_ref[...], wl_ref[...], preferred_element_type=jnp.float32)
        + jnp.dot(h_ref[...], wr_ref[...], preferred_element_type=jnp.float32)
        + b2_ref[...])
    h3 = jax.nn.relu(
        jnp.dot(h2, w4_ref[...], preferred_element_type=jnp.float32)
        + b4_ref[...])
    o_ref[...] = (jnp.dot(h3, w5_ref[...], preferred_element_type=jnp.float32)
                  + b5_ref[...])


def kernel(state, edge_index, W1l, W1r, b1, W2l, W2r, b2, W4, b4, W5, b5):
    src = edge_index[0]
    dst = edge_index[1]
    xp = jnp.pad(state, ((0, NP - N), (0, 0)))

    agg1, lsrc, ldl, nb = _seg_scan(xp, src, dst)
    agg1 = agg1.reshape(NP, D)

    W1lp = jnp.pad(W1l, ((0, 0), (0, 2)))
    W1rp = jnp.pad(W1r, ((0, 0), (0, 2)))
    b1p = jnp.pad(b1, (0, 2)).reshape(1, 16)
    h1 = pl.pallas_call(
        _tc1_body,
        out_shape=jax.ShapeDtypeStruct((NP, 16), jnp.float32),
    )(agg1, xp, W1lp, W1rp, b1p)

    agg2 = _seg_replay(h1, lsrc, ldl, nb).reshape(NP, 16)

    W2lp = jnp.pad(W2l, ((0, 2), (0, 0)))
    W2rp = jnp.pad(W2r, ((0, 2), (0, 0)))
    out = pl.pallas_call(
        _tc2_body,
        out_shape=jax.ShapeDtypeStruct((NP, 1), jnp.float32),
    )(agg2, h1, W2lp, W2rp, b2.reshape(1, 8), W4, b4.reshape(1, 5), W5,
      b5.reshape(1, 1))

    return out[:N, 0]
